# Initial kernel scaffold; baseline (speedup 1.0000x reference)
import jax
import jax.numpy as jnp
from jax.experimental import pallas as pl


def kernel(x, adj, weights_nbrs, weights_self, bias):
    raise NotImplementedError("write your pallas kernel here")


# speedup vs baseline: 3.4601x; 3.4601x over previous
import functools

import jax
import jax.numpy as jnp
from jax import lax
from jax.experimental import pallas as pl
from jax.experimental.pallas import tpu as pltpu
from jax.experimental.pallas import tpu_sc as plsc

NC = 2
NS = 16
EDGE_B = 128
ZROWS = 64
ROW_B = 1000


def _round_up(v, m):
    return (v + m - 1) // m * m


def _matmul_body(x_ref, w_ref, y_ref):
    y_ref[...] = lax.dot_general(
        x_ref[...], w_ref[...], (((1,), (0,)), ((), ())),
        precision=lax.Precision.HIGHEST,
        preferred_element_type=jnp.float32)


def _combine_body(p_ref, x_ref, w_ref, b_ref, o_ref):
    s = lax.dot_general(
        x_ref[...], w_ref[...], (((1,), (0,)), ((), ())),
        precision=lax.Precision.HIGHEST,
        preferred_element_type=jnp.float32)
    o_ref[...] = p_ref[0] + s + b_ref[...]


def _make_sc_segment_sum(n_acc, d, nb):
    per = n_acc // NS
    mesh = plsc.VectorSubcoreMesh(core_axis_name="c", subcore_axis_name="s")

    @functools.partial(
        pl.kernel,
        out_type=jax.ShapeDtypeStruct((NC, n_acc, d), jnp.float32),
        mesh=mesh,
        scratch_types=[
            pltpu.VMEM((nb, EDGE_B), jnp.int32),
            pltpu.VMEM((nb, EDGE_B), jnp.int32),
            pltpu.VMEM((EDGE_B, d), jnp.float32),
            pltpu.VMEM((EDGE_B, d), jnp.float32),
            pltpu.VMEM((ZROWS, d), jnp.float32),
            pltpu.VMEM_SHARED((n_acc, d), jnp.float32),
            pltpu.SemaphoreType.DMA,
            pltpu.SemaphoreType.DMA,
        ],
    )
    def sc_segsum(y_hbm, colv_hbm, rowv_hbm, out_hbm,
                  col_v, row_v, buf0, buf1, zbuf, acc, sem0, sem1):
        cid = lax.axis_index("c")
        sid = lax.axis_index("s")

        @pl.loop(0, ZROWS)
        def _(r):
            @pl.loop(0, d, step=16)
            def _(c):
                zbuf[r, pl.ds(c, 16)] = jnp.zeros((16,), jnp.float32)

        @pl.loop(0, per, step=ZROWS)
        def _(k):
            pltpu.sync_copy(zbuf, acc.at[pl.ds(sid * per + k, ZROWS)])

        pltpu.sync_copy(colv_hbm.at[sid], col_v)
        pltpu.sync_copy(rowv_hbm.at[cid, sid], row_v)

        plsc.subcore_barrier()

        pltpu.async_copy(y_hbm.at[col_v.at[0]], buf0, sem0)

        @pl.loop(0, nb, step=2)
        def _(j):
            pltpu.make_async_copy(y_hbm.at[col_v.at[j]], buf0, sem0).wait()
            pltpu.async_copy(y_hbm.at[col_v.at[j + 1]], buf1, sem1)
            pltpu.sync_copy(buf0, acc.at[row_v.at[j]], add=True)
            pltpu.make_async_copy(y_hbm.at[col_v.at[j + 1]], buf1, sem1).wait()

            @pl.when(j + 2 < nb)
            def _():
                pltpu.async_copy(y_hbm.at[col_v.at[j + 2]], buf0, sem0)

            pltpu.sync_copy(buf1, acc.at[row_v.at[j + 1]], add=True)

        plsc.subcore_barrier()

        pltpu.sync_copy(acc.at[pl.ds(sid * per, per)],
                        out_hbm.at[cid, pl.ds(sid * per, per)])

    return sc_segsum


def kernel(x, adj, weights_nbrs, weights_self, bias):
    n, d_in = x.shape
    d_out = weights_nbrs.shape[1]
    e = adj.shape[1]
    half = n // 2

    epw = _round_up(-(-e // NS), 2 * EDGE_B)
    e_pad = epw * NS
    nb = epw // EDGE_B
    n_acc = _round_up(half + 1, NS * ZROWS)

    adj32 = adj.astype(jnp.int32)
    pad = e_pad - e
    colv = jnp.concatenate([adj32[1], jnp.zeros((pad,), jnp.int32)])
    rowd = jnp.concatenate([adj32[0], jnp.full((pad,), n, jnp.int32)])
    row0 = jnp.where(rowd < half, rowd, half)
    row1 = jnp.where((rowd >= half) & (rowd < n), rowd - half, half)
    colv = colv.reshape(NS, nb, EDGE_B)
    rowv = jnp.stack([row0, row1]).reshape(NC, NS, nb, EDGE_B)

    grid = n // ROW_B
    hb = half // ROW_B
    y = pl.pallas_call(
        _matmul_body,
        grid=(grid,),
        in_specs=[
            pl.BlockSpec((ROW_B, d_in), lambda i: (i, 0)),
            pl.BlockSpec((d_in, d_out), lambda i: (0, 0)),
        ],
        out_specs=pl.BlockSpec((ROW_B, d_out), lambda i: (i, 0)),
        out_shape=jax.ShapeDtypeStruct((n, d_out), jnp.float32),
    )(x, weights_nbrs)

    partials = _make_sc_segment_sum(n_acc, d_out, nb)(y, colv, rowv)

    out = pl.pallas_call(
        _combine_body,
        grid=(grid,),
        in_specs=[
            pl.BlockSpec((1, ROW_B, d_out), lambda i: (i // hb, i % hb, 0)),
            pl.BlockSpec((ROW_B, d_in), lambda i: (i, 0)),
            pl.BlockSpec((d_in, d_out), lambda i: (0, 0)),
            pl.BlockSpec((1, d_out), lambda i: (0, 0)),
        ],
        out_specs=pl.BlockSpec((ROW_B, d_out), lambda i: (i, 0)),
        out_shape=jax.ShapeDtypeStruct((n, d_out), jnp.float32),
    )(partials, x, weights_self, bias.reshape(1, d_out))

    return out


# speedup vs baseline: 3.5572x; 1.0281x over previous
import functools

import jax
import jax.numpy as jnp
from jax import lax
from jax.experimental import pallas as pl
from jax.experimental.pallas import tpu as pltpu
from jax.experimental.pallas import tpu_sc as plsc

NC = 2
NS = 16
EDGE_B = 128
ZROWS = 64
ROW_B = 1000


def _round_up(v, m):
    return (v + m - 1) // m * m


def _matmul_body(x_ref, w_ref, y_ref):
    y_ref[...] = lax.dot_general(
        x_ref[...], w_ref[...], (((1,), (0,)), ((), ())),
        precision=lax.Precision.HIGHEST,
        preferred_element_type=jnp.float32)


def _combine_body(p_ref, x_ref, w_ref, b_ref, o_ref):
    s = lax.dot_general(
        x_ref[...], w_ref[...], (((1,), (0,)), ((), ())),
        precision=lax.Precision.HIGHEST,
        preferred_element_type=jnp.float32)
    o_ref[...] = p_ref[0] + s + b_ref[...]


def _make_sc_segment_sum(n_acc, d, nb):
    per = n_acc // NS
    mesh = plsc.VectorSubcoreMesh(core_axis_name="c", subcore_axis_name="s")

    @functools.partial(
        pl.kernel,
        out_type=jax.ShapeDtypeStruct((NC, n_acc, d), jnp.float32),
        mesh=mesh,
        scratch_types=[
            pltpu.VMEM((nb, EDGE_B), jnp.int32),
            pltpu.VMEM((nb, EDGE_B), jnp.int32),
            pltpu.VMEM((EDGE_B, d), jnp.float32),
            pltpu.VMEM((EDGE_B, d), jnp.float32),
            pltpu.VMEM((ZROWS, d), jnp.float32),
            pltpu.VMEM_SHARED((n_acc, d), jnp.float32),
            pltpu.SemaphoreType.DMA,
            pltpu.SemaphoreType.DMA,
        ],
    )
    def sc_segsum(y_hbm, colv_hbm, rowv_hbm, out_hbm,
                  col_v, row_v, buf0, buf1, zbuf, acc, sem0, sem1):
        cid = lax.axis_index("c")
        sid = lax.axis_index("s")

        @pl.loop(0, ZROWS)
        def _(r):
            @pl.loop(0, d, step=16)
            def _(c):
                zbuf[r, pl.ds(c, 16)] = jnp.zeros((16,), jnp.float32)

        @pl.loop(0, per, step=ZROWS)
        def _(k):
            pltpu.sync_copy(zbuf, acc.at[pl.ds(sid * per + k, ZROWS)])

        pltpu.sync_copy(colv_hbm.at[sid], col_v)
        pltpu.sync_copy(rowv_hbm.at[cid, sid], row_v)

        plsc.subcore_barrier()

        pltpu.async_copy(y_hbm.at[col_v.at[0]], buf0, sem0)

        @pl.loop(0, nb, step=2)
        def _(j):
            pltpu.make_async_copy(y_hbm.at[col_v.at[j]], buf0, sem0).wait()
            pltpu.async_copy(y_hbm.at[col_v.at[j + 1]], buf1, sem1)
            pltpu.sync_copy(buf0, acc.at[row_v.at[j]], add=True)
            pltpu.make_async_copy(y_hbm.at[col_v.at[j + 1]], buf1, sem1).wait()

            @pl.when(j + 2 < nb)
            def _():
                pltpu.async_copy(y_hbm.at[col_v.at[j + 2]], buf0, sem0)

            pltpu.sync_copy(buf1, acc.at[row_v.at[j + 1]], add=True)

        plsc.subcore_barrier()

        pltpu.sync_copy(acc.at[pl.ds(sid * per, per)],
                        out_hbm.at[cid, pl.ds(sid * per, per)])

    return sc_segsum


def kernel(x, adj, weights_nbrs, weights_self, bias):
    n, d_in = x.shape
    d_out = weights_nbrs.shape[1]
    e = adj.shape[1]
    half = n // 2

    epw = _round_up(-(-e // NS), 2 * EDGE_B)
    e_pad = epw * NS
    nb = epw // EDGE_B
    n_acc = _round_up(half + 1, NS * ZROWS)

    adj32 = adj.astype(jnp.int32)
    pad = e_pad - e
    colv = jnp.concatenate([adj32[1], jnp.zeros((pad,), jnp.int32)])
    rowd = jnp.concatenate([adj32[0], jnp.full((pad,), n, jnp.int32)])
    spread = (jnp.arange(e_pad, dtype=jnp.int32) & 1023) + half
    row0 = jnp.where(rowd < half, rowd, spread)
    row1 = jnp.where((rowd >= half) & (rowd < n), rowd - half, spread)
    colv = colv.reshape(NS, nb, EDGE_B)
    rowv = jnp.stack([row0, row1]).reshape(NC, NS, nb, EDGE_B)

    grid = n // ROW_B
    hb = half // ROW_B
    y = pl.pallas_call(
        _matmul_body,
        grid=(grid,),
        in_specs=[
            pl.BlockSpec((ROW_B, d_in), lambda i: (i, 0)),
            pl.BlockSpec((d_in, d_out), lambda i: (0, 0)),
        ],
        out_specs=pl.BlockSpec((ROW_B, d_out), lambda i: (i, 0)),
        out_shape=jax.ShapeDtypeStruct((n, d_out), jnp.float32),
    )(x, weights_nbrs)

    partials = _make_sc_segment_sum(n_acc, d_out, nb)(y, colv, rowv)

    out = pl.pallas_call(
        _combine_body,
        grid=(grid,),
        in_specs=[
            pl.BlockSpec((1, ROW_B, d_out), lambda i: (i // hb, i % hb, 0)),
            pl.BlockSpec((ROW_B, d_in), lambda i: (i, 0)),
            pl.BlockSpec((d_in, d_out), lambda i: (0, 0)),
            pl.BlockSpec((1, d_out), lambda i: (0, 0)),
        ],
        out_specs=pl.BlockSpec((ROW_B, d_out), lambda i: (i, 0)),
        out_shape=jax.ShapeDtypeStruct((n, d_out), jnp.float32),
    )(partials, x, weights_self, bias.reshape(1, d_out))

    return out


# speedup vs baseline: 5.2967x; 1.4890x over previous
import dataclasses
import functools

import jax
import jax.numpy as jnp
from jax import lax
from jax.experimental import pallas as pl
from jax.experimental.pallas import tpu as pltpu
from jax.experimental.pallas import tpu_sc as plsc

NC = 2
NS = 16
EDGE_B = 128
ZROWS = 32
ROW_B = 1000


def _round_up(v, m):
    return (v + m - 1) // m * m


def _matmul_body(x_ref, w_ref, y_ref):
    y_ref[...] = lax.dot_general(
        x_ref[...], w_ref[...], (((1,), (0,)), ((), ())),
        precision=lax.Precision.HIGHEST,
        preferred_element_type=jnp.float32)


def _combine_body(p_ref, x_ref, w_ref, b_ref, o_ref):
    s = lax.dot_general(
        x_ref[...], w_ref[...], (((1,), (0,)), ((), ())),
        precision=lax.Precision.HIGHEST,
        preferred_element_type=jnp.float32)
    o_ref[...] = p_ref[0] + s + b_ref[...]


def _make_sc_segment_sum(n_acc, d, epw, half):
    per = n_acc // NS
    cap = epw + 2 * EDGE_B
    mesh = plsc.VectorSubcoreMesh(core_axis_name="c", subcore_axis_name="s")
    cp = pltpu.CompilerParams()
    if "needs_layout_passes" in pltpu.CompilerParams.__dataclass_fields__:
        cp = dataclasses.replace(cp, needs_layout_passes=False)

    @functools.partial(
        pl.kernel,
        out_type=jax.ShapeDtypeStruct((NC, n_acc, d), jnp.float32),
        mesh=mesh,
        compiler_params=cp,
        scratch_types=[
            pltpu.VMEM((cap,), jnp.int32),
            pltpu.VMEM((cap,), jnp.int32),
            pltpu.VMEM((1, EDGE_B), jnp.int32),
            pltpu.VMEM((EDGE_B, d), jnp.float32),
            pltpu.VMEM((EDGE_B, d), jnp.float32),
            pltpu.VMEM((ZROWS, d), jnp.float32),
            pltpu.VMEM_SHARED((n_acc, d), jnp.float32),
            pltpu.SemaphoreType.DMA,
            pltpu.SemaphoreType.DMA,
        ],
    )
    def sc_segsum(y_hbm, colv_hbm, rowv_hbm, out_hbm,
                  col_v, row_v, ridx2, buf0, buf1, zbuf, acc, sem0, sem1):
        cid = lax.axis_index("c")
        sid = lax.axis_index("s")
        lo = cid * half

        @pl.loop(0, ZROWS)
        def _(r):
            @pl.loop(0, d, step=16)
            def _(c):
                zbuf[r, pl.ds(c, 16)] = jnp.zeros((16,), jnp.float32)

        @pl.loop(0, per, step=ZROWS)
        def _(k):
            pltpu.sync_copy(zbuf, acc.at[pl.ds(sid * per + k, ZROWS)])

        pltpu.sync_copy(colv_hbm.at[pl.ds(sid * epw, epw)],
                        col_v.at[pl.ds(0, epw)])
        pltpu.sync_copy(rowv_hbm.at[pl.ds(sid * epw, epw)],
                        row_v.at[pl.ds(0, epw)])

        def compact_step(i, q):
            p = i * 16
            r = row_v[pl.ds(p, 16)]
            c = col_v[pl.ds(p, 16)]
            mask = (r >= lo) & (r < lo + half)
            plsc.store_compressed(row_v.at[pl.ds(q, 16)], r - lo, mask=mask)
            plsc.store_compressed(col_v.at[pl.ds(q, 16)], c, mask=mask)
            return q + jnp.sum(mask.astype(jnp.int32))

        q = lax.fori_loop(0, epw // 16, compact_step, jnp.int32(0))

        lanes = lax.iota(jnp.int32, 16)

        @pl.loop(0, 2 * EDGE_B, step=16)
        def _(k):
            col_v[pl.ds(q + k, 16)] = jnp.zeros((16,), jnp.int32)
            row_v[pl.ds(q + k, 16)] = half + k + lanes

        nb2 = q // (2 * EDGE_B) + 1

        plsc.subcore_barrier()

        def scatter(j, buf):
            @pl.loop(0, EDGE_B, step=16)
            def _(c):
                ridx2[0, pl.ds(c, 16)] = row_v[pl.ds(j * EDGE_B + c, 16)]
            pltpu.sync_copy(buf, acc.at[ridx2.at[0]], add=True)

        def gather_start(j, buf, sem):
            pltpu.async_copy(
                y_hbm.at[col_v.at[pl.ds(j * EDGE_B, EDGE_B)]], buf, sem)

        def gather_wait(j, buf, sem):
            pltpu.make_async_copy(
                y_hbm.at[col_v.at[pl.ds(j * EDGE_B, EDGE_B)]], buf, sem).wait()

        gather_start(0, buf0, sem0)

        def pair_step(i, carry):
            j = i * 2
            gather_wait(j, buf0, sem0)
            gather_start(j + 1, buf1, sem1)
            scatter(j, buf0)
            gather_wait(j + 1, buf1, sem1)

            @pl.when(j + 2 < nb2 * 2)
            def _():
                gather_start(j + 2, buf0, sem0)

            scatter(j + 1, buf1)
            return carry

        lax.fori_loop(0, nb2, pair_step, jnp.int32(0))

        plsc.subcore_barrier()

        pltpu.sync_copy(acc.at[pl.ds(sid * per, per)],
                        out_hbm.at[cid, pl.ds(sid * per, per)])

    return sc_segsum


def kernel(x, adj, weights_nbrs, weights_self, bias):
    n, d_in = x.shape
    d_out = weights_nbrs.shape[1]
    e = adj.shape[1]
    half = n // 2

    epw = _round_up(-(-e // NS), 16)
    e_pad = epw * NS
    n_acc = _round_up(half + 2 * EDGE_B + 16, NS * ZROWS)

    adj32 = adj.astype(jnp.int32)
    pad = e_pad - e
    if pad:
        colv = jnp.concatenate([adj32[1], jnp.zeros((pad,), jnp.int32)])
        rowv = jnp.concatenate([adj32[0], jnp.full((pad,), n, jnp.int32)])
    else:
        colv, rowv = adj32[1], adj32[0]

    grid = n // ROW_B
    hb = half // ROW_B
    y = pl.pallas_call(
        _matmul_body,
        grid=(grid,),
        in_specs=[
            pl.BlockSpec((ROW_B, d_in), lambda i: (i, 0)),
            pl.BlockSpec((d_in, d_out), lambda i: (0, 0)),
        ],
        out_specs=pl.BlockSpec((ROW_B, d_out), lambda i: (i, 0)),
        out_shape=jax.ShapeDtypeStruct((n, d_out), jnp.float32),
    )(x, weights_nbrs)

    partials = _make_sc_segment_sum(n_acc, d_out, epw, half)(y, colv, rowv)

    out = pl.pallas_call(
        _combine_body,
        grid=(grid,),
        in_specs=[
            pl.BlockSpec((1, ROW_B, d_out), lambda i: (i // hb, i % hb, 0)),
            pl.BlockSpec((ROW_B, d_in), lambda i: (i, 0)),
            pl.BlockSpec((d_in, d_out), lambda i: (0, 0)),
            pl.BlockSpec((1, d_out), lambda i: (0, 0)),
        ],
        out_specs=pl.BlockSpec((ROW_B, d_out), lambda i: (i, 0)),
        out_shape=jax.ShapeDtypeStruct((n, d_out), jnp.float32),
    )(partials, x, weights_self, bias.reshape(1, d_out))

    return out


# speedup vs baseline: 5.4916x; 1.0368x over previous
import dataclasses
import functools

import jax
import jax.numpy as jnp
from jax import lax
from jax.experimental import pallas as pl
from jax.experimental.pallas import tpu as pltpu
from jax.experimental.pallas import tpu_sc as plsc

NC = 2
NS = 16
EDGE_B = 64
NBUF = 4
ZROWS = 32
ROW_B = 1000


def _round_up(v, m):
    return (v + m - 1) // m * m


def _matmul_body(x_ref, w_ref, y_ref):
    y_ref[...] = lax.dot_general(
        x_ref[...], w_ref[...], (((1,), (0,)), ((), ())),
        precision=lax.Precision.HIGHEST,
        preferred_element_type=jnp.float32)


def _combine_body(p_ref, x_ref, w_ref, b_ref, o_ref):
    s = lax.dot_general(
        x_ref[...], w_ref[...], (((1,), (0,)), ((), ())),
        precision=lax.Precision.HIGHEST,
        preferred_element_type=jnp.float32)
    o_ref[...] = p_ref[0] + s + b_ref[...]


def _make_sc_segment_sum(n_acc, d, epw, half):
    per = n_acc // NS
    cap = epw + NBUF * EDGE_B
    mesh = plsc.VectorSubcoreMesh(core_axis_name="c", subcore_axis_name="s")
    cp = pltpu.CompilerParams()
    if "needs_layout_passes" in pltpu.CompilerParams.__dataclass_fields__:
        cp = dataclasses.replace(cp, needs_layout_passes=False)

    @functools.partial(
        pl.kernel,
        out_type=jax.ShapeDtypeStruct((NC, n_acc, d), jnp.float32),
        mesh=mesh,
        compiler_params=cp,
        scratch_types=[
            pltpu.VMEM((cap,), jnp.int32),
            pltpu.VMEM((cap,), jnp.int32),
            pltpu.VMEM((NBUF, EDGE_B), jnp.int32),
        ] + [pltpu.VMEM((EDGE_B, d), jnp.float32) for _ in range(NBUF)] + [
            pltpu.VMEM((ZROWS, d), jnp.float32),
            pltpu.VMEM_SHARED((n_acc, d), jnp.float32),
        ] + [pltpu.SemaphoreType.DMA for _ in range(2 * NBUF)],
    )
    def sc_segsum(y_hbm, colv_hbm, rowv_hbm, out_hbm,
                  col_v, row_v, ridx2, *rest):
        bufs = rest[:NBUF]
        zbuf = rest[NBUF]
        acc = rest[NBUF + 1]
        gsems = rest[NBUF + 2:2 * NBUF + 2]
        ssems = rest[2 * NBUF + 2:]
        cid = lax.axis_index("c")
        sid = lax.axis_index("s")
        lo = cid * half

        @pl.loop(0, ZROWS)
        def _(r):
            @pl.loop(0, d, step=16)
            def _(c):
                zbuf[r, pl.ds(c, 16)] = jnp.zeros((16,), jnp.float32)

        @pl.loop(0, per, step=ZROWS)
        def _(k):
            pltpu.sync_copy(zbuf, acc.at[pl.ds(sid * per + k, ZROWS)])

        pltpu.sync_copy(colv_hbm.at[pl.ds(sid * epw, epw)],
                        col_v.at[pl.ds(0, epw)])
        pltpu.sync_copy(rowv_hbm.at[pl.ds(sid * epw, epw)],
                        row_v.at[pl.ds(0, epw)])

        def compact_step(i, q):
            p = i * 16
            r = row_v[pl.ds(p, 16)]
            c = col_v[pl.ds(p, 16)]
            mask = (r >= lo) & (r < lo + half)
            plsc.store_compressed(row_v.at[pl.ds(q, 16)], r - lo, mask=mask)
            plsc.store_compressed(col_v.at[pl.ds(q, 16)], c, mask=mask)
            return q + plsc.all_reduce_population_count(mask)[0]

        q = lax.fori_loop(0, epw // 16, compact_step, jnp.int32(0))

        lanes = lax.iota(jnp.int32, 16)

        @pl.loop(0, NBUF * EDGE_B, step=16)
        def _(k):
            col_v[pl.ds(q + k, 16)] = jnp.zeros((16,), jnp.int32)
            row_v[pl.ds(q + k, 16)] = half + k + lanes

        nbq = q // (NBUF * EDGE_B) + 1
        nb = nbq * NBUF

        plsc.subcore_barrier()

        def scatter_start(j, k):
            @pl.loop(0, EDGE_B, step=16)
            def _(c):
                ridx2[k, pl.ds(c, 16)] = row_v[pl.ds(j * EDGE_B + c, 16)]
            pltpu.async_copy(bufs[k], acc.at[ridx2.at[k]], ssems[k], add=True)

        def scatter_wait(k):
            pltpu.make_async_copy(bufs[k], acc.at[ridx2.at[k]],
                                  ssems[k]).wait()

        def gather_start(j, k):
            pltpu.async_copy(
                y_hbm.at[col_v.at[pl.ds(j * EDGE_B, EDGE_B)]], bufs[k],
                gsems[k])

        def gather_wait(j, k):
            pltpu.make_async_copy(
                y_hbm.at[col_v.at[pl.ds(j * EDGE_B, EDGE_B)]], bufs[k],
                gsems[k]).wait()

        for k in range(NBUF):
            gather_start(k, k)

        def ring_step(i, carry):
            j = i * NBUF
            for k in range(NBUF):
                gather_wait(j + k, k)
                scatter_start(j + k, k)
            for k in range(NBUF):
                scatter_wait(k)

                @pl.when(j + k + NBUF < nb)
                def _(k=k, j=j):
                    gather_start(j + k + NBUF, k)
            return carry

        lax.fori_loop(0, nbq, ring_step, jnp.int32(0))

        plsc.subcore_barrier()

        pltpu.sync_copy(acc.at[pl.ds(sid * per, per)],
                        out_hbm.at[cid, pl.ds(sid * per, per)])

    return sc_segsum


def kernel(x, adj, weights_nbrs, weights_self, bias):
    n, d_in = x.shape
    d_out = weights_nbrs.shape[1]
    e = adj.shape[1]
    half = n // 2

    epw = _round_up(-(-e // NS), 16)
    e_pad = epw * NS
    n_acc = _round_up(half + NBUF * EDGE_B + 16, NS * ZROWS)

    adj32 = adj.astype(jnp.int32)
    pad = e_pad - e
    if pad:
        colv = jnp.concatenate([adj32[1], jnp.zeros((pad,), jnp.int32)])
        rowv = jnp.concatenate([adj32[0], jnp.full((pad,), n, jnp.int32)])
    else:
        colv, rowv = adj32[1], adj32[0]

    grid = n // ROW_B
    hb = half // ROW_B
    y = pl.pallas_call(
        _matmul_body,
        grid=(grid,),
        in_specs=[
            pl.BlockSpec((ROW_B, d_in), lambda i: (i, 0)),
            pl.BlockSpec((d_in, d_out), lambda i: (0, 0)),
        ],
        out_specs=pl.BlockSpec((ROW_B, d_out), lambda i: (i, 0)),
        out_shape=jax.ShapeDtypeStruct((n, d_out), jnp.float32),
    )(x, weights_nbrs)

    partials = _make_sc_segment_sum(n_acc, d_out, epw, half)(y, colv, rowv)

    out = pl.pallas_call(
        _combine_body,
        grid=(grid,),
        in_specs=[
            pl.BlockSpec((1, ROW_B, d_out), lambda i: (i // hb, i % hb, 0)),
            pl.BlockSpec((ROW_B, d_in), lambda i: (i, 0)),
            pl.BlockSpec((d_in, d_out), lambda i: (0, 0)),
            pl.BlockSpec((1, d_out), lambda i: (0, 0)),
        ],
        out_specs=pl.BlockSpec((ROW_B, d_out), lambda i: (i, 0)),
        out_shape=jax.ShapeDtypeStruct((n, d_out), jnp.float32),
    )(partials, x, weights_self, bias.reshape(1, d_out))

    return out


# speedup vs baseline: 5.6449x; 1.0279x over previous
import dataclasses
import functools

import jax
import jax.numpy as jnp
from jax import lax
from jax.experimental import pallas as pl
from jax.experimental.pallas import tpu as pltpu
from jax.experimental.pallas import tpu_sc as plsc

NC = 2
NS = 16
EDGE_B = 64
NBUF = 3
ZROWS = 32
ROW_B = 1000


def _round_up(v, m):
    return (v + m - 1) // m * m


def _matmul_body(x_ref, w_ref, y_ref):
    y_ref[...] = lax.dot_general(
        x_ref[...], w_ref[...], (((1,), (0,)), ((), ())),
        precision=lax.Precision.HIGHEST,
        preferred_element_type=jnp.float32).astype(jnp.bfloat16)


def _combine_body(p_ref, x_ref, w_ref, b_ref, o_ref):
    s = lax.dot_general(
        x_ref[...], w_ref[...], (((1,), (0,)), ((), ())),
        precision=lax.Precision.HIGHEST,
        preferred_element_type=jnp.float32)
    o_ref[...] = p_ref[0] + s + b_ref[...]


def _make_sc_segment_sum(n_acc, d, epw, half):
    per = n_acc // NS
    cap = epw + NBUF * EDGE_B
    dh = d // 2
    mesh = plsc.VectorSubcoreMesh(core_axis_name="c", subcore_axis_name="s")
    cp = pltpu.CompilerParams()
    for fld, val in (("needs_layout_passes", False),
                     ("use_tc_tiling_on_sc", False)):
        if fld in pltpu.CompilerParams.__dataclass_fields__:
            cp = dataclasses.replace(cp, **{fld: val})

    @functools.partial(
        pl.kernel,
        out_type=jax.ShapeDtypeStruct((NC, n_acc, d), jnp.float32),
        mesh=mesh,
        compiler_params=cp,
        scratch_types=[
            pltpu.VMEM((cap,), jnp.int32),
            pltpu.VMEM((cap,), jnp.int32),
            pltpu.VMEM((NBUF, EDGE_B), jnp.int32),
        ] + [pltpu.VMEM((EDGE_B, dh), jnp.int32) for _ in range(NBUF)] + [
            pltpu.VMEM((EDGE_B, d), jnp.float32) for _ in range(NBUF)] + [
            pltpu.VMEM((ZROWS, d), jnp.float32),
            pltpu.VMEM_SHARED((n_acc, d), jnp.float32),
        ] + [pltpu.SemaphoreType.DMA for _ in range(2 * NBUF)],
    )
    def sc_segsum(y_hbm, colv_hbm, rowv_hbm, out_hbm,
                  col_v, row_v, ridx2, *rest):
        gbufs = rest[:NBUF]
        fbufs = rest[NBUF:2 * NBUF]
        zbuf = rest[2 * NBUF]
        acc = rest[2 * NBUF + 1]
        gsems = rest[2 * NBUF + 2:3 * NBUF + 2]
        ssems = rest[3 * NBUF + 2:]
        cid = lax.axis_index("c")
        sid = lax.axis_index("s")
        lo = cid * half

        @pl.loop(0, ZROWS)
        def _(r):
            @pl.loop(0, d, step=16)
            def _(c):
                zbuf[r, pl.ds(c, 16)] = jnp.zeros((16,), jnp.float32)

        @pl.loop(0, per, step=ZROWS)
        def _(k):
            pltpu.sync_copy(zbuf, acc.at[pl.ds(sid * per + k, ZROWS)])

        pltpu.sync_copy(colv_hbm.at[pl.ds(sid * epw, epw)],
                        col_v.at[pl.ds(0, epw)])
        pltpu.sync_copy(rowv_hbm.at[pl.ds(sid * epw, epw)],
                        row_v.at[pl.ds(0, epw)])

        def compact_step(i, q):
            p = i * 16
            r = row_v[pl.ds(p, 16)]
            c = col_v[pl.ds(p, 16)]
            mask = (r >= lo) & (r < lo + half)
            plsc.store_compressed(row_v.at[pl.ds(q, 16)], r - lo, mask=mask)
            plsc.store_compressed(col_v.at[pl.ds(q, 16)], c, mask=mask)
            return q + plsc.all_reduce_population_count(mask)[0]

        q = lax.fori_loop(0, epw // 16, compact_step, jnp.int32(0))

        lanes = lax.iota(jnp.int32, 16)

        @pl.loop(0, NBUF * EDGE_B, step=16)
        def _(k):
            col_v[pl.ds(q + k, 16)] = jnp.zeros((16,), jnp.int32)
            row_v[pl.ds(q + k, 16)] = half + k + lanes

        nbq = q // (NBUF * EDGE_B) + 1
        nb = nbq * NBUF

        plsc.subcore_barrier()

        def scatter_start(j, k):
            @pl.loop(0, EDGE_B, step=16)
            def _(c):
                ridx2[k, pl.ds(c, 16)] = row_v[pl.ds(j * EDGE_B + c, 16)]
            pltpu.async_copy(fbufs[k], acc.at[ridx2.at[k]], ssems[k], add=True)

        def scatter_wait(k):
            pltpu.make_async_copy(fbufs[k], acc.at[ridx2.at[k]],
                                  ssems[k]).wait()

        def gather_start(j, k):
            pltpu.async_copy(
                y_hbm.at[col_v.at[pl.ds(j * EDGE_B, EDGE_B)]], gbufs[k],
                gsems[k])

        def gather_wait(j, k):
            pltpu.make_async_copy(
                y_hbm.at[col_v.at[pl.ds(j * EDGE_B, EDGE_B)]], gbufs[k],
                gsems[k]).wait()

        def convert(k):
            hi_mask = jnp.full((16,), -65536, jnp.int32)

            @pl.loop(0, EDGE_B)
            def _(r):
                for c in range(0, dh, 16):
                    v = gbufs[k][r, pl.ds(c, 16)]
                    lo = plsc.bitcast(v << 16, jnp.float32)
                    hi = plsc.bitcast(v & hi_mask, jnp.float32)
                    fbufs[k][r, pl.ds(c, 16)] = lo
                    fbufs[k][r, pl.ds(dh + c, 16)] = hi

        for k in range(NBUF):
            gather_start(k, k)

        def ring_step(i, carry):
            j = i * NBUF
            for k in range(NBUF):
                gather_wait(j + k, k)

                @pl.when(j + k >= NBUF)
                def _(k=k):
                    scatter_wait(k)

                convert(k)
                scatter_start(j + k, k)

                @pl.when(j + k + NBUF < nb)
                def _(k=k, j=j):
                    gather_start(j + k + NBUF, k)
            return carry

        lax.fori_loop(0, nbq, ring_step, jnp.int32(0))

        for k in range(NBUF):
            scatter_wait(k)

        plsc.subcore_barrier()

        pltpu.sync_copy(acc.at[pl.ds(sid * per, per)],
                        out_hbm.at[cid, pl.ds(sid * per, per)])

    return sc_segsum


def kernel(x, adj, weights_nbrs, weights_self, bias):
    n, d_in = x.shape
    d_out = weights_nbrs.shape[1]
    e = adj.shape[1]
    half = n // 2

    epw = _round_up(-(-e // NS), 16)
    e_pad = epw * NS
    n_acc = _round_up(half + NBUF * EDGE_B + 16, NS * ZROWS)

    adj32 = adj.astype(jnp.int32)
    pad = e_pad - e
    if pad:
        colv = jnp.concatenate([adj32[1], jnp.zeros((pad,), jnp.int32)])
        rowv = jnp.concatenate([adj32[0], jnp.full((pad,), n, jnp.int32)])
    else:
        colv, rowv = adj32[1], adj32[0]

    grid = n // ROW_B
    hb = half // ROW_B
    y = pl.pallas_call(
        _matmul_body,
        grid=(grid,),
        in_specs=[
            pl.BlockSpec((ROW_B, d_in), lambda i: (i, 0)),
            pl.BlockSpec((d_in, d_out), lambda i: (0, 0)),
        ],
        out_specs=pl.BlockSpec((ROW_B, d_out), lambda i: (i, 0)),
        out_shape=jax.ShapeDtypeStruct((n, d_out), jnp.bfloat16),
    )(x, weights_nbrs)

    dh = d_out // 2
    y32 = lax.bitcast_convert_type(
        jnp.stack([y[:, :dh], y[:, dh:]], axis=2), jnp.int32)

    partials = _make_sc_segment_sum(n_acc, d_out, epw, half)(y32, colv, rowv)

    out = pl.pallas_call(
        _combine_body,
        grid=(grid,),
        in_specs=[
            pl.BlockSpec((1, ROW_B, d_out), lambda i: (i // hb, i % hb, 0)),
            pl.BlockSpec((ROW_B, d_in), lambda i: (i, 0)),
            pl.BlockSpec((d_in, d_out), lambda i: (0, 0)),
            pl.BlockSpec((1, d_out), lambda i: (0, 0)),
        ],
        out_specs=pl.BlockSpec((ROW_B, d_out), lambda i: (i, 0)),
        out_shape=jax.ShapeDtypeStruct((n, d_out), jnp.float32),
    )(partials, x, weights_self, bias.reshape(1, d_out))

    return out


# speedup vs baseline: 5.7224x; 1.0137x over previous
import dataclasses
import functools

import jax
import jax.numpy as jnp
from jax import lax
from jax.experimental import pallas as pl
from jax.experimental.pallas import tpu as pltpu
from jax.experimental.pallas import tpu_sc as plsc

NC = 2
NS = 16
EDGE_B = 64
NBUF = 3
ZROWS = 32
ROW_B = 1000


def _round_up(v, m):
    return (v + m - 1) // m * m


def _matmul_body(x_ref, w_ref, y_ref):
    y_ref[...] = lax.dot_general(
        x_ref[...], w_ref[...], (((1,), (0,)), ((), ())),
        precision=lax.Precision.HIGHEST,
        preferred_element_type=jnp.float32).astype(jnp.bfloat16)


def _self_body(x_ref, w_ref, b_ref, s_ref):
    s_ref[...] = lax.dot_general(
        x_ref[...], w_ref[...], (((1,), (0,)), ((), ())),
        precision=lax.Precision.HIGHEST,
        preferred_element_type=jnp.float32) + b_ref[...]


def _combine_body(p_ref, s_ref, o_ref):
    o_ref[...] = p_ref[0] + s_ref[...]


def _make_sc_segment_sum(n_acc, d, epw, half):
    per = n_acc // NS
    cap = epw + NBUF * EDGE_B
    dh = d // 2
    mesh = plsc.VectorSubcoreMesh(core_axis_name="c", subcore_axis_name="s")
    cp = pltpu.CompilerParams()
    for fld, val in (("needs_layout_passes", False),
                     ("use_tc_tiling_on_sc", False)):
        if fld in pltpu.CompilerParams.__dataclass_fields__:
            cp = dataclasses.replace(cp, **{fld: val})

    @functools.partial(
        pl.kernel,
        out_type=jax.ShapeDtypeStruct((NC, n_acc, d), jnp.float32),
        mesh=mesh,
        compiler_params=cp,
        scratch_types=[
            pltpu.VMEM((cap,), jnp.int32),
            pltpu.VMEM((cap,), jnp.int32),
            pltpu.VMEM((NBUF, EDGE_B), jnp.int32),
        ] + [pltpu.VMEM((EDGE_B, dh), jnp.int32) for _ in range(NBUF)] + [
            pltpu.VMEM((EDGE_B, d), jnp.float32) for _ in range(NBUF)] + [
            pltpu.VMEM((ZROWS, d), jnp.float32),
            pltpu.VMEM_SHARED((n_acc, d), jnp.float32),
        ] + [pltpu.SemaphoreType.DMA for _ in range(2 * NBUF)],
    )
    def sc_segsum(y_hbm, colv_hbm, rowv_hbm, out_hbm,
                  col_v, row_v, ridx2, *rest):
        gbufs = rest[:NBUF]
        fbufs = rest[NBUF:2 * NBUF]
        zbuf = rest[2 * NBUF]
        acc = rest[2 * NBUF + 1]
        gsems = rest[2 * NBUF + 2:3 * NBUF + 2]
        ssems = rest[3 * NBUF + 2:]
        cid = lax.axis_index("c")
        sid = lax.axis_index("s")
        lo = cid * half

        @pl.loop(0, ZROWS)
        def _(r):
            @pl.loop(0, d, step=16)
            def _(c):
                zbuf[r, pl.ds(c, 16)] = jnp.zeros((16,), jnp.float32)

        @pl.loop(0, per, step=ZROWS)
        def _(k):
            pltpu.sync_copy(zbuf, acc.at[pl.ds(sid * per + k, ZROWS)])

        pltpu.sync_copy(colv_hbm.at[pl.ds(sid * epw, epw)],
                        col_v.at[pl.ds(0, epw)])
        pltpu.sync_copy(rowv_hbm.at[pl.ds(sid * epw, epw)],
                        row_v.at[pl.ds(0, epw)])

        def compact_step(i, q):
            p = i * 16
            r = row_v[pl.ds(p, 16)]
            c = col_v[pl.ds(p, 16)]
            mask = (r >= lo) & (r < lo + half)
            plsc.store_compressed(row_v.at[pl.ds(q, 16)], r - lo, mask=mask)
            plsc.store_compressed(col_v.at[pl.ds(q, 16)], c, mask=mask)
            return q + plsc.all_reduce_population_count(mask)[0]

        q = lax.fori_loop(0, epw // 16, compact_step, jnp.int32(0))

        lanes = lax.iota(jnp.int32, 16)

        @pl.loop(0, NBUF * EDGE_B, step=16)
        def _(k):
            col_v[pl.ds(q + k, 16)] = jnp.zeros((16,), jnp.int32)
            row_v[pl.ds(q + k, 16)] = half + k + lanes

        nbq = q // (NBUF * EDGE_B) + 1
        nb = nbq * NBUF

        plsc.subcore_barrier()

        def scatter_start(j, k):
            @pl.loop(0, EDGE_B, step=16)
            def _(c):
                ridx2[k, pl.ds(c, 16)] = row_v[pl.ds(j * EDGE_B + c, 16)]
            pltpu.async_copy(fbufs[k], acc.at[ridx2.at[k]], ssems[k], add=True)

        def scatter_wait(k):
            pltpu.make_async_copy(fbufs[k], acc.at[ridx2.at[k]],
                                  ssems[k]).wait()

        def gather_start(j, k):
            pltpu.async_copy(
                y_hbm.at[col_v.at[pl.ds(j * EDGE_B, EDGE_B)]], gbufs[k],
                gsems[k])

        def gather_wait(j, k):
            pltpu.make_async_copy(
                y_hbm.at[col_v.at[pl.ds(j * EDGE_B, EDGE_B)]], gbufs[k],
                gsems[k]).wait()

        def convert(k):
            hi_mask = jnp.full((16,), -65536, jnp.int32)

            @pl.loop(0, EDGE_B)
            def _(r):
                for c in range(0, dh, 16):
                    v = gbufs[k][r, pl.ds(c, 16)]
                    lo = plsc.bitcast(v << 16, jnp.float32)
                    hi = plsc.bitcast(v & hi_mask, jnp.float32)
                    fbufs[k][r, pl.ds(c, 16)] = lo
                    fbufs[k][r, pl.ds(dh + c, 16)] = hi

        for k in range(NBUF):
            gather_start(k, k)

        def ring_step(i, carry):
            j = i * NBUF
            for k in range(NBUF):
                gather_wait(j + k, k)

                @pl.when(j + k >= NBUF)
                def _(k=k):
                    scatter_wait(k)

                convert(k)
                scatter_start(j + k, k)

                @pl.when(j + k + NBUF < nb)
                def _(k=k, j=j):
                    gather_start(j + k + NBUF, k)
            return carry

        lax.fori_loop(0, nbq, ring_step, jnp.int32(0))

        for k in range(NBUF):
            scatter_wait(k)

        plsc.subcore_barrier()

        pltpu.sync_copy(acc.at[pl.ds(sid * per, per)],
                        out_hbm.at[cid, pl.ds(sid * per, per)])

    return sc_segsum


def kernel(x, adj, weights_nbrs, weights_self, bias):
    n, d_in = x.shape
    d_out = weights_nbrs.shape[1]
    e = adj.shape[1]
    half = n // 2

    epw = _round_up(-(-e // NS), 16)
    e_pad = epw * NS
    n_acc = _round_up(half + NBUF * EDGE_B + 16, NS * ZROWS)

    adj32 = adj.astype(jnp.int32)
    pad = e_pad - e
    if pad:
        colv = jnp.concatenate([adj32[1], jnp.zeros((pad,), jnp.int32)])
        rowv = jnp.concatenate([adj32[0], jnp.full((pad,), n, jnp.int32)])
    else:
        colv, rowv = adj32[1], adj32[0]

    grid = n // ROW_B
    hb = half // ROW_B
    y = pl.pallas_call(
        _matmul_body,
        grid=(grid,),
        in_specs=[
            pl.BlockSpec((ROW_B, d_in), lambda i: (i, 0)),
            pl.BlockSpec((d_in, d_out), lambda i: (0, 0)),
        ],
        out_specs=pl.BlockSpec((ROW_B, d_out), lambda i: (i, 0)),
        out_shape=jax.ShapeDtypeStruct((n, d_out), jnp.bfloat16),
    )(x, weights_nbrs)

    dh = d_out // 2
    y32 = lax.bitcast_convert_type(
        jnp.stack([y[:, :dh], y[:, dh:]], axis=2), jnp.int32)

    s = pl.pallas_call(
        _self_body,
        grid=(grid,),
        in_specs=[
            pl.BlockSpec((ROW_B, d_in), lambda i: (i, 0)),
            pl.BlockSpec((d_in, d_out), lambda i: (0, 0)),
            pl.BlockSpec((1, d_out), lambda i: (0, 0)),
        ],
        out_specs=pl.BlockSpec((ROW_B, d_out), lambda i: (i, 0)),
        out_shape=jax.ShapeDtypeStruct((n, d_out), jnp.float32),
    )(x, weights_self, bias.reshape(1, d_out))

    partials = _make_sc_segment_sum(n_acc, d_out, epw, half)(y32, colv, rowv)

    out = pl.pallas_call(
        _combine_body,
        grid=(grid,),
        in_specs=[
            pl.BlockSpec((1, ROW_B, d_out), lambda i: (i // hb, i % hb, 0)),
            pl.BlockSpec((ROW_B, d_out), lambda i: (i, 0)),
        ],
        out_specs=pl.BlockSpec((ROW_B, d_out), lambda i: (i, 0)),
        out_shape=jax.ShapeDtypeStruct((n, d_out), jnp.float32),
    )(partials, s)

    return out
